# baseline (speedup 1.0000x reference)
import jax
import jax.numpy as jnp
from jax.experimental import pallas as pl
from jax.experimental.pallas import tpu as pltpu

_COUNTS = 12
_ROWS = 36
_TOK = 77
_DIM = 512
_HEAD = 8
_BULK = _TOK - _HEAD
_NQ = 8


def _body(pos_ref, fh_ref, h_ref, a_ref, f_any, out_hbm, buf, hsem, sems):
    pos0 = pos_ref[0]
    pos1 = pos_ref[1]
    fh = fh_ref[...]
    h = h_ref[...]
    a = a_ref[...]
    buf[...] = jnp.broadcast_to(
        fh[:, None], (3, _COUNTS, _HEAD, _DIM)).reshape(_ROWS, _HEAD, _DIM)
    h36 = jnp.broadcast_to(h[None, :, None], (3, 3, 4, 1, _DIM)).reshape(
        _ROWS, 1, _DIM)
    a36 = jnp.broadcast_to(a[None], (9, 4, 1, _DIM)).reshape(_ROWS, 1, _DIM)
    buf[:, pl.ds(pos0, 1), :] = h36
    buf[:, pl.ds(pos1, 1), :] = a36
    copies = []
    for i in range(_ROWS):
        fi = i // _COUNTS
        copies.append(pltpu.make_async_copy(
            f_any.at[fi, pl.ds(_HEAD, _BULK)],
            out_hbm.at[i, pl.ds(_HEAD, _BULK)],
            sems.at[i % _NQ]))
    head_copy = pltpu.make_async_copy(
        buf, out_hbm.at[:, pl.ds(0, _HEAD)], hsem)
    head_copy.start()
    for c in copies:
        c.start()
    head_copy.wait()
    for c in copies:
        c.wait()


def kernel(freeze_embedding, height_param, angle_param, pos0, pos1):
    posv = jnp.stack([jnp.asarray(pos0, jnp.int32),
                      jnp.asarray(pos1, jnp.int32)])
    return pl.pallas_call(
        _body,
        grid=(1,),
        in_specs=[
            pl.BlockSpec(memory_space=pltpu.SMEM),
            pl.BlockSpec((3, _HEAD, _DIM), lambda i: (0, 0, 0)),
            pl.BlockSpec(memory_space=pltpu.VMEM),
            pl.BlockSpec(memory_space=pltpu.VMEM),
            pl.BlockSpec(memory_space=pl.ANY),
        ],
        out_specs=pl.BlockSpec(memory_space=pl.ANY),
        out_shape=jax.ShapeDtypeStruct((_ROWS, _TOK, _DIM), jnp.float32),
        scratch_shapes=[
            pltpu.VMEM((_ROWS, _HEAD, _DIM), jnp.float32),
            pltpu.SemaphoreType.DMA,
            pltpu.SemaphoreType.DMA((_NQ,)),
        ],
    )(posv, freeze_embedding,
      height_param.reshape(3, 1, _DIM), angle_param.reshape(4, 1, _DIM),
      freeze_embedding)


# speedup vs baseline: 19.8394x; 19.8394x over previous
import jax
import jax.numpy as jnp
from jax import lax
from jax.experimental import pallas as pl
from jax.experimental.pallas import tpu as pltpu

_COUNTS = 12
_ROWS = 36
_TOK = 77
_DIM = 512


def _body(pos_ref, f_ref, h_ref, a_ref, out_ref):
    pos0 = pos_ref[0]
    pos1 = pos_ref[1]
    f = f_ref[...]
    h = h_ref[...]
    a = a_ref[...]
    h12 = jnp.broadcast_to(h[:, None], (3, 4, 1, _DIM)).reshape(12, 1, _DIM)
    a12 = jnp.broadcast_to(a[None], (3, 4, 1, _DIM)).reshape(12, 1, _DIM)
    out_ref[...] = jnp.broadcast_to(f, (_COUNTS, _TOK, _DIM))
    out_ref[:, pl.ds(pos0, 1), :] = h12
    out_ref[:, pl.ds(pos1, 1), :] = a12


def kernel(freeze_embedding, height_param, angle_param, pos0, pos1):
    posv = jnp.stack([jnp.asarray(pos0, jnp.int32),
                      jnp.asarray(pos1, jnp.int32)])
    return pl.pallas_call(
        _body,
        grid=(3,),
        in_specs=[
            pl.BlockSpec(memory_space=pltpu.SMEM),
            pl.BlockSpec((1, _TOK, _DIM), lambda i: (i, 0, 0)),
            pl.BlockSpec((3, 1, _DIM), lambda i: (0, 0, 0)),
            pl.BlockSpec((4, 1, _DIM), lambda i: (0, 0, 0)),
        ],
        out_specs=pl.BlockSpec((_COUNTS, _TOK, _DIM), lambda i: (i, 0, 0)),
        out_shape=jax.ShapeDtypeStruct((_ROWS, _TOK, _DIM), jnp.float32),
    )(posv, freeze_embedding,
      height_param.reshape(3, 1, _DIM), angle_param.reshape(4, 1, _DIM))
